# baseline (speedup 1.0000x reference)
import numpy as np

import jax
import jax.numpy as jnp
from jax.experimental import pallas as pl
from jax.experimental.pallas import tpu as pltpu

EPS = 1e-12
H = 16
W = 900
C = 128
K = 64
N_WIN = 30
WIN_CHUNKS = 20
STEP_CHUNKS = 3
CHUNK = 10 * H
NG = 90
NG_PAD = 96
BLK = 3

_WSEL = np.zeros((N_WIN, NG_PAD), np.float32)
for _j in range(N_WIN):
    for _r in range(STEP_CHUNKS * _j, STEP_CHUNKS * _j + WIN_CHUNKS):
        _WSEL[_j, (_r - 10) % NG] += 1.0
_WSEL.setflags(write=False)


def _vlad_kernel(wsel_ref, xt_ref, cw_ref, cb_ref, dcb_ref, out_ref,
                 sbuf, abuf, wbuf):
    cw = cw_ref[...]
    cb = cb_ref[...]
    sbuf[NG:NG_PAD] = jnp.zeros((NG_PAD - NG, K, C), jnp.bfloat16)

    for b in range(NG // BLK):
        xb = xt_ref[0, b * BLK * CHUNK:(b + 1) * BLK * CHUNK, :].astype(
            jnp.bfloat16)
        lb = jnp.dot(xb, cw, preferred_element_type=jnp.float32) + cb
        e = jnp.exp(lb)
        ab = e * (1.0 / jnp.sum(e, axis=1, keepdims=True))
        abuf[b] = jnp.sum(ab.reshape(BLK, CHUNK, K), axis=1)
        abh = ab.astype(jnp.bfloat16)
        for u in range(BLK):
            g = b * BLK + u
            ag = abh[u * CHUNK:(u + 1) * CHUNK]
            xg = xb[u * CHUNK:(u + 1) * CHUNK]
            sbuf[g] = jax.lax.dot_general(
                ag, xg, (((0,), (0,)), ((), ())),
                preferred_element_type=jnp.float32).astype(jnp.bfloat16)

    ws = wsel_ref[...]
    av = jnp.concatenate(
        [abuf[...].reshape(NG, K),
         jnp.zeros((NG_PAD - NG, K), jnp.float32)], axis=0)
    wa = jax.lax.dot_general(
        av.astype(jnp.bfloat16), ws, (((0,), (1,)), ((), ())),
        preferred_element_type=jnp.float32)
    wsx = jnp.concatenate(
        [ws, -jnp.transpose(wa).astype(jnp.bfloat16)], axis=1)
    sv = jnp.concatenate([sbuf[...], dcb_ref[...]], axis=0)
    wbuf[...] = jax.lax.dot_general(
        wsx, sv, (((1,), (0,)), ((), ())),
        preferred_element_type=jnp.float32)

    for j in range(N_WIN):
        v = wbuf[j]
        ss = jnp.sum(v * v, axis=1, keepdims=True)
        inv = 1.0 / jnp.maximum(jnp.sqrt(ss), EPS)
        g2 = jnp.sum(ss * inv * inv, axis=0, keepdims=True)
        ginv = 1.0 / jnp.maximum(jnp.sqrt(g2), EPS)
        out_ref[0, j] = (v * (inv * ginv)).astype(jnp.bfloat16)


def _mlp_kernel(v_ref, w_ref, b_ref, o_ref):
    kk = pl.program_id(0)
    z = jax.lax.dot_general(
        v_ref[...], w_ref[...].astype(jnp.bfloat16),
        (((1,), (1,)), ((), ())),
        preferred_element_type=jnp.float32)

    @pl.when(kk == 0)
    def _():
        o_ref[...] = z

    @pl.when(kk > 0)
    def _():
        o_ref[...] += z

    @pl.when(kk == 3)
    def _():
        zf = o_ref[...] + b_ref[...]
        ss = jnp.sum(zf * zf, axis=1, keepdims=True)
        o_ref[...] = zf * (1.0 / jnp.maximum(jnp.sqrt(ss), EPS))


def kernel(x, cluster_centers, conv_w, conv_b, mlp_w, mlp_b):
    n = x.shape[0]
    rows = W * H
    xt = jnp.transpose(x, (0, 3, 2, 1)).reshape(n, rows, C)
    cw = conv_w.T.astype(jnp.bfloat16)
    cb = conv_b.reshape(1, K)
    wsel = jnp.asarray(_WSEL).astype(jnp.bfloat16)
    dcb = (jnp.eye(K, dtype=jnp.float32)[:, :, None]
           * cluster_centers[None, :, :]).astype(jnp.bfloat16)

    vlad = pl.pallas_call(
        _vlad_kernel,
        out_shape=jax.ShapeDtypeStruct((n, N_WIN, K, C), jnp.bfloat16),
        grid=(n,),
        in_specs=[
            pl.BlockSpec((N_WIN, NG_PAD), lambda i: (0, 0)),
            pl.BlockSpec((1, rows, C), lambda i: (i, 0, 0)),
            pl.BlockSpec((C, K), lambda i: (0, 0)),
            pl.BlockSpec((1, K), lambda i: (0, 0)),
            pl.BlockSpec((K, K, C), lambda i: (0, 0, 0)),
        ],
        out_specs=pl.BlockSpec((1, N_WIN, K, C), lambda i: (i, 0, 0, 0)),
        scratch_shapes=[
            pltpu.VMEM((NG_PAD, K, C), jnp.bfloat16),
            pltpu.VMEM((NG // BLK, BLK, K), jnp.float32),
            pltpu.VMEM((N_WIN, K, C), jnp.float32),
        ],
        compiler_params=pltpu.CompilerParams(
            dimension_semantics=("arbitrary",),
        ),
        name="yaw_netvlad_windows",
    )(wsel, xt, cw, cb, dcb)

    vflat = vlad.reshape(n * N_WIN, K * C)
    kc = K * C // 4
    out = pl.pallas_call(
        _mlp_kernel,
        out_shape=jax.ShapeDtypeStruct((n * N_WIN, mlp_w.shape[0]), jnp.float32),
        grid=(4,),
        in_specs=[
            pl.BlockSpec((n * N_WIN, kc), lambda k: (0, k)),
            pl.BlockSpec((mlp_w.shape[0], kc), lambda k: (0, k)),
            pl.BlockSpec((1, mlp_w.shape[0]), lambda k: (0, 0)),
        ],
        out_specs=pl.BlockSpec((n * N_WIN, mlp_w.shape[0]), lambda k: (0, 0)),
        compiler_params=pltpu.CompilerParams(
            dimension_semantics=("arbitrary",),
        ),
        name="yaw_netvlad_mlp",
    )(vflat, mlp_w, mlp_b.reshape(1, -1))

    return out.reshape(n, N_WIN, mlp_w.shape[0])


# speedup vs baseline: 1.0145x; 1.0145x over previous
import numpy as np

import jax
import jax.numpy as jnp
from jax.experimental import pallas as pl
from jax.experimental.pallas import tpu as pltpu

EPS = 1e-12
H = 16
W = 900
C = 128
K = 64
N_WIN = 30
WIN_CHUNKS = 20
STEP_CHUNKS = 3
CHUNK = 10 * H
NG = 90
NG_PAD = 96
BLK = 5

_WSEL = np.zeros((N_WIN, NG_PAD), np.float32)
for _j in range(N_WIN):
    for _r in range(STEP_CHUNKS * _j, STEP_CHUNKS * _j + WIN_CHUNKS):
        _WSEL[_j, (_r - 10) % NG] += 1.0
_WSEL.setflags(write=False)


def _vlad_kernel(wsel_ref, xt_ref, cw_ref, cb_ref, dcb_ref, out_ref,
                 sbuf, abuf, wbuf):
    cw = cw_ref[...]
    cb = cb_ref[...]
    sbuf[NG:NG_PAD] = jnp.zeros((NG_PAD - NG, K, C), jnp.bfloat16)

    for b in range(NG // BLK):
        xb = xt_ref[0, b * BLK * CHUNK:(b + 1) * BLK * CHUNK, :].astype(
            jnp.bfloat16)
        lb = jnp.dot(xb, cw, preferred_element_type=jnp.float32) + cb
        e = jnp.exp(lb)
        ab = e * (1.0 / jnp.sum(e, axis=1, keepdims=True))
        abuf[b] = jnp.sum(ab.reshape(BLK, CHUNK, K), axis=1)
        abh = ab.astype(jnp.bfloat16)
        for u in range(BLK):
            g = b * BLK + u
            ag = abh[u * CHUNK:(u + 1) * CHUNK]
            xg = xb[u * CHUNK:(u + 1) * CHUNK]
            sbuf[g] = jax.lax.dot_general(
                ag, xg, (((0,), (0,)), ((), ())),
                preferred_element_type=jnp.float32).astype(jnp.bfloat16)

    ws = wsel_ref[...]
    av = jnp.concatenate(
        [abuf[...].reshape(NG, K),
         jnp.zeros((NG_PAD - NG, K), jnp.float32)], axis=0)
    wa = jax.lax.dot_general(
        av.astype(jnp.bfloat16), ws, (((0,), (1,)), ((), ())),
        preferred_element_type=jnp.float32)
    wsx = jnp.concatenate(
        [ws, -jnp.transpose(wa).astype(jnp.bfloat16)], axis=1)
    sv = jnp.concatenate([sbuf[...], dcb_ref[...]], axis=0)
    wbuf[...] = jax.lax.dot_general(
        wsx, sv, (((1,), (0,)), ((), ())),
        preferred_element_type=jnp.float32)

    for j in range(N_WIN):
        v = wbuf[j]
        ss = jnp.sum(v * v, axis=1, keepdims=True)
        inv = 1.0 / jnp.maximum(jnp.sqrt(ss), EPS)
        g2 = jnp.sum(ss * inv * inv, axis=0, keepdims=True)
        ginv = 1.0 / jnp.maximum(jnp.sqrt(g2), EPS)
        out_ref[0, j] = (v * (inv * ginv)).astype(jnp.bfloat16)


def _mlp_kernel(v_ref, w_ref, b_ref, o_ref):
    kk = pl.program_id(0)
    z = jax.lax.dot_general(
        v_ref[...], w_ref[...].astype(jnp.bfloat16),
        (((1,), (1,)), ((), ())),
        preferred_element_type=jnp.float32)

    @pl.when(kk == 0)
    def _():
        o_ref[...] = z

    @pl.when(kk > 0)
    def _():
        o_ref[...] += z

    @pl.when(kk == 3)
    def _():
        zf = o_ref[...] + b_ref[...]
        ss = jnp.sum(zf * zf, axis=1, keepdims=True)
        o_ref[...] = zf * (1.0 / jnp.maximum(jnp.sqrt(ss), EPS))


def kernel(x, cluster_centers, conv_w, conv_b, mlp_w, mlp_b):
    n = x.shape[0]
    rows = W * H
    xt = jnp.transpose(x, (0, 3, 2, 1)).reshape(n, rows, C)
    cw = conv_w.T.astype(jnp.bfloat16)
    cb = conv_b.reshape(1, K)
    wsel = jnp.asarray(_WSEL).astype(jnp.bfloat16)
    dcb = (jnp.eye(K, dtype=jnp.float32)[:, :, None]
           * cluster_centers[None, :, :]).astype(jnp.bfloat16)

    vlad = pl.pallas_call(
        _vlad_kernel,
        out_shape=jax.ShapeDtypeStruct((n, N_WIN, K, C), jnp.bfloat16),
        grid=(n,),
        in_specs=[
            pl.BlockSpec((N_WIN, NG_PAD), lambda i: (0, 0)),
            pl.BlockSpec((1, rows, C), lambda i: (i, 0, 0)),
            pl.BlockSpec((C, K), lambda i: (0, 0)),
            pl.BlockSpec((1, K), lambda i: (0, 0)),
            pl.BlockSpec((K, K, C), lambda i: (0, 0, 0)),
        ],
        out_specs=pl.BlockSpec((1, N_WIN, K, C), lambda i: (i, 0, 0, 0)),
        scratch_shapes=[
            pltpu.VMEM((NG_PAD, K, C), jnp.bfloat16),
            pltpu.VMEM((NG // BLK, BLK, K), jnp.float32),
            pltpu.VMEM((N_WIN, K, C), jnp.float32),
        ],
        compiler_params=pltpu.CompilerParams(
            dimension_semantics=("arbitrary",),
        ),
        name="yaw_netvlad_windows",
    )(wsel, xt, cw, cb, dcb)

    vflat = vlad.reshape(n * N_WIN, K * C)
    kc = K * C // 4
    out = pl.pallas_call(
        _mlp_kernel,
        out_shape=jax.ShapeDtypeStruct((n * N_WIN, mlp_w.shape[0]), jnp.float32),
        grid=(4,),
        in_specs=[
            pl.BlockSpec((n * N_WIN, kc), lambda k: (0, k)),
            pl.BlockSpec((mlp_w.shape[0], kc), lambda k: (0, k)),
            pl.BlockSpec((1, mlp_w.shape[0]), lambda k: (0, 0)),
        ],
        out_specs=pl.BlockSpec((n * N_WIN, mlp_w.shape[0]), lambda k: (0, 0)),
        compiler_params=pltpu.CompilerParams(
            dimension_semantics=("arbitrary",),
        ),
        name="yaw_netvlad_mlp",
    )(vflat, mlp_w, mlp_b.reshape(1, -1))

    return out.reshape(n, N_WIN, mlp_w.shape[0])
